# baseline (speedup 1.0000x reference)
import functools

import jax
import jax.numpy as jnp
from jax import lax
from jax.experimental import pallas as pl
from jax.experimental.pallas import tpu as pltpu
from jax.experimental.pallas import tpu_sc as plsc

BATCH = 16384
HIDDEN = 32
NUM_CORES = 2
NUM_SUBCORES = 16
NUM_WORKERS = NUM_CORES * NUM_SUBCORES
B_PER_W = BATCH // NUM_WORKERS
LANES = 16


def _mf_body(user_hbm, item_hbm, uft_hbm, ift_hbm, out_hbm,
             uidx_v, iidx_v, ucols_v, icols_v, out_v, sem):
    wid = lax.axis_index("s") * NUM_CORES + lax.axis_index("c")
    base = wid * B_PER_W

    pltpu.sync_copy(user_hbm.at[pl.ds(base, B_PER_W)], uidx_v)
    pltpu.sync_copy(item_hbm.at[pl.ds(base, B_PER_W)], iidx_v)

    for d in range(HIDDEN):
        pltpu.async_copy(uft_hbm.at[d].at[uidx_v],
                         ucols_v.at[pl.ds(d * B_PER_W, B_PER_W)], sem)
        pltpu.async_copy(ift_hbm.at[d].at[iidx_v],
                         icols_v.at[pl.ds(d * B_PER_W, B_PER_W)], sem)

    pltpu.make_async_copy(uft_hbm.at[0].at[pl.ds(0, HIDDEN * B_PER_W)],
                          ucols_v, sem).wait()
    pltpu.make_async_copy(ift_hbm.at[0].at[pl.ds(0, HIDDEN * B_PER_W)],
                          icols_v, sem).wait()

    def chunk(c, _):
        acc = jnp.zeros((LANES,), jnp.float32)
        for d in range(HIDDEN):
            u = ucols_v[pl.ds(d * B_PER_W + c * LANES, LANES)]
            v = icols_v[pl.ds(d * B_PER_W + c * LANES, LANES)]
            acc = acc + u * v
        out_v[pl.ds(c * LANES, LANES)] = acc
        return 0

    lax.fori_loop(0, B_PER_W // LANES, chunk, 0)

    pltpu.sync_copy(out_v, out_hbm.at[pl.ds(base, B_PER_W)])


_mf = functools.partial(
    pl.kernel,
    out_type=jax.ShapeDtypeStruct((BATCH,), jnp.float32),
    mesh=plsc.VectorSubcoreMesh(core_axis_name="c", subcore_axis_name="s"),
    scratch_types=[
        pltpu.VMEM((B_PER_W,), jnp.int32),
        pltpu.VMEM((B_PER_W,), jnp.int32),
        pltpu.VMEM((HIDDEN * B_PER_W,), jnp.float32),
        pltpu.VMEM((HIDDEN * B_PER_W,), jnp.float32),
        pltpu.VMEM((B_PER_W,), jnp.float32),
        pltpu.SemaphoreType.DMA,
    ],
    compiler_params=pltpu.CompilerParams(use_tc_tiling_on_sc=False),
)(_mf_body)


def kernel(user, item, user_factors, item_factors):
    return _mf(user.astype(jnp.int32), item.astype(jnp.int32),
               user_factors.T, item_factors.T)


# speedup vs baseline: 22.5661x; 22.5661x over previous
import functools

import jax
import jax.numpy as jnp
from jax import lax
from jax.experimental import pallas as pl
from jax.experimental.pallas import tpu as pltpu
from jax.experimental.pallas import tpu_sc as plsc

BATCH = 16384
HIDDEN = 32
NUM_CORES = 2
NUM_SUBCORES = 16
NUM_WORKERS = NUM_CORES * NUM_SUBCORES
B_PER_W = BATCH // NUM_WORKERS
LANES = 16
RING = 8


def _mf_body(user_hbm, item_hbm, uft_hbm, ift_hbm, out_hbm,
             uidx_v, iidx_v, ubuf_v, ibuf_v, out_v, sem_u, sem_i):
    wid = lax.axis_index("s") * NUM_CORES + lax.axis_index("c")
    base = wid * B_PER_W

    pltpu.sync_copy(user_hbm.at[pl.ds(base, B_PER_W)], uidx_v)
    pltpu.sync_copy(item_hbm.at[pl.ds(base, B_PER_W)], iidx_v)

    lane = lax.iota(jnp.int32, LANES)

    def idx_scalar(idx_ref, b):
        vec = idx_ref[pl.ds((b >> 4) << 4, LANES)]
        return jnp.sum(jnp.where(lane == (b & 15), vec, 0))

    def fetch(b, slot):
        cu = pl.multiple_of((idx_scalar(uidx_v, b) >> 7) * 128, 128)
        ci = pl.multiple_of((idx_scalar(iidx_v, b) >> 7) * 128, 128)
        pltpu.async_copy(uft_hbm.at[:, pl.ds(cu, 128)], ubuf_v.at[slot],
                         sem_u.at[slot])
        pltpu.async_copy(ift_hbm.at[:, pl.ds(ci, 128)], ibuf_v.at[slot],
                         sem_i.at[slot])

    for r in range(RING):
        fetch(r, r)

    d_lo = lane
    d_hi = lane + LANES

    def round_body(g, _):
        for r in range(RING):
            b = g * RING + r
            pltpu.make_async_copy(
                uft_hbm.at[:, pl.ds(0, 128)], ubuf_v.at[r], sem_u.at[r]).wait()
            pltpu.make_async_copy(
                ift_hbm.at[:, pl.ds(0, 128)], ibuf_v.at[r], sem_i.at[r]).wait()
            lu = jnp.full((LANES,), idx_scalar(uidx_v, b) & 127, jnp.int32)
            li = jnp.full((LANES,), idx_scalar(iidx_v, b) & 127, jnp.int32)
            rv = jnp.full((LANES,), r, jnp.int32)
            u_lo = plsc.load_gather(ubuf_v, [rv, d_lo, lu])
            u_hi = plsc.load_gather(ubuf_v, [rv, d_hi, lu])
            i_lo = plsc.load_gather(ibuf_v, [rv, d_lo, li])
            i_hi = plsc.load_gather(ibuf_v, [rv, d_hi, li])
            val = jnp.sum(u_lo * i_lo + u_hi * i_hi)
            plsc.store_scatter(out_v, [jnp.full((LANES,), b, jnp.int32)],
                               jnp.full((LANES,), val, jnp.float32),
                               mask=lane == 0)
            nb = b + RING

            @pl.when(nb < B_PER_W)
            def _():
                fetch(nb, r)

        return 0

    lax.fori_loop(0, B_PER_W // RING, round_body, 0)

    pltpu.sync_copy(out_v, out_hbm.at[pl.ds(base, B_PER_W)])


_mf = functools.partial(
    pl.kernel,
    out_type=jax.ShapeDtypeStruct((BATCH,), jnp.float32),
    mesh=plsc.VectorSubcoreMesh(core_axis_name="c", subcore_axis_name="s"),
    scratch_types=[
        pltpu.VMEM((B_PER_W,), jnp.int32),
        pltpu.VMEM((B_PER_W,), jnp.int32),
        pltpu.VMEM((RING, HIDDEN, 128), jnp.float32),
        pltpu.VMEM((RING, HIDDEN, 128), jnp.float32),
        pltpu.VMEM((B_PER_W,), jnp.float32),
        pltpu.SemaphoreType.DMA((RING,)),
        pltpu.SemaphoreType.DMA((RING,)),
    ],
    compiler_params=pltpu.CompilerParams(needs_layout_passes=False),
)(_mf_body)


def kernel(user, item, user_factors, item_factors):
    return _mf(user.astype(jnp.int32), item.astype(jnp.int32),
               user_factors.T, item_factors.T)


# speedup vs baseline: 22.5913x; 1.0011x over previous
import functools

import jax
import jax.numpy as jnp
from jax import lax
from jax.experimental import pallas as pl
from jax.experimental.pallas import tpu as pltpu
from jax.experimental.pallas import tpu_sc as plsc

BATCH = 16384
HIDDEN = 32
NUM_CORES = 2
NUM_SUBCORES = 16
NUM_WORKERS = NUM_CORES * NUM_SUBCORES
B_PER_W = BATCH // NUM_WORKERS
LANES = 16
RING = 8


def _mf_body(user_hbm, item_hbm, uft_hbm, ift_hbm, out_hbm,
             uidx_v, iidx_v, ubuf_v, ibuf_v, out_v, sem_u, sem_i):
    wid = lax.axis_index("s") * NUM_CORES + lax.axis_index("c")
    base = wid * B_PER_W

    pltpu.sync_copy(user_hbm.at[pl.ds(base, B_PER_W)], uidx_v)
    pltpu.sync_copy(item_hbm.at[pl.ds(base, B_PER_W)], iidx_v)

    lane = lax.iota(jnp.int32, LANES)

    def idx_scalar(idx_ref, b):
        vec = idx_ref[pl.ds((b >> 4) << 4, LANES)]
        return jnp.sum(jnp.where(lane == (b & 15), vec, 0))

    def fetch(b, slot):
        cu = pl.multiple_of((idx_scalar(uidx_v, b) >> 7) * 128, 128)
        ci = pl.multiple_of((idx_scalar(iidx_v, b) >> 7) * 128, 128)
        pltpu.async_copy(uft_hbm.at[:, pl.ds(cu, 128)], ubuf_v.at[slot],
                         sem_u.at[slot])
        pltpu.async_copy(ift_hbm.at[:, pl.ds(ci, 128)], ibuf_v.at[slot],
                         sem_i.at[slot])

    for r in range(RING):
        fetch(r, r)

    d_lo = lane
    d_hi = lane + LANES

    def round_body(g, _):
        for r in range(RING):
            b = g * RING + r
            pltpu.make_async_copy(
                uft_hbm.at[:, pl.ds(0, 128)], ubuf_v.at[r], sem_u.at[r]).wait()
            pltpu.make_async_copy(
                ift_hbm.at[:, pl.ds(0, 128)], ibuf_v.at[r], sem_i.at[r]).wait()
            lu = jnp.full((LANES,), idx_scalar(uidx_v, b) & 127, jnp.int32)
            li = jnp.full((LANES,), idx_scalar(iidx_v, b) & 127, jnp.int32)
            rv = jnp.full((LANES,), r, jnp.int32)
            u_lo = plsc.load_gather(ubuf_v, [rv, d_lo, lu])
            u_hi = plsc.load_gather(ubuf_v, [rv, d_hi, lu])
            i_lo = plsc.load_gather(ibuf_v, [rv, d_lo, li])
            i_hi = plsc.load_gather(ibuf_v, [rv, d_hi, li])
            nb = b + RING

            @pl.when(nb < B_PER_W)
            def _():
                fetch(nb, r)

            val = jnp.sum(u_lo * i_lo + u_hi * i_hi)
            plsc.store_scatter(out_v, [jnp.full((LANES,), b, jnp.int32)],
                               jnp.full((LANES,), val, jnp.float32),
                               mask=lane == 0)

        return 0

    lax.fori_loop(0, B_PER_W // RING, round_body, 0)

    pltpu.sync_copy(out_v, out_hbm.at[pl.ds(base, B_PER_W)])


_mf = functools.partial(
    pl.kernel,
    out_type=jax.ShapeDtypeStruct((BATCH,), jnp.float32),
    mesh=plsc.VectorSubcoreMesh(core_axis_name="c", subcore_axis_name="s"),
    scratch_types=[
        pltpu.VMEM((B_PER_W,), jnp.int32),
        pltpu.VMEM((B_PER_W,), jnp.int32),
        pltpu.VMEM((RING, HIDDEN, 128), jnp.float32),
        pltpu.VMEM((RING, HIDDEN, 128), jnp.float32),
        pltpu.VMEM((B_PER_W,), jnp.float32),
        pltpu.SemaphoreType.DMA((RING,)),
        pltpu.SemaphoreType.DMA((RING,)),
    ],
    compiler_params=pltpu.CompilerParams(needs_layout_passes=False),
)(_mf_body)


def kernel(user, item, user_factors, item_factors):
    return _mf(user.astype(jnp.int32), item.astype(jnp.int32),
               user_factors.T, item_factors.T)
